# Initial kernel scaffold; baseline (speedup 1.0000x reference)
import jax
import jax.numpy as jnp
from jax.experimental import pallas as pl


def kernel(x, token_table, pos_table):
    raise NotImplementedError("write your pallas kernel here")


# speedup vs baseline: 1.0291x; 1.0291x over previous
import functools

import jax
import jax.numpy as jnp
from jax import lax
from jax.experimental import pallas as pl
from jax.experimental.pallas import tpu as pltpu
from jax.experimental.pallas import tpu_sc as plsc


_LANES = 16


@functools.cache
def _build(num_tokens: int, seq_len: int, d_model: int):
    info = plsc.get_sparse_core_info()
    nc, ns = info.num_cores, info.num_subcores
    nw = nc * ns
    per_w = num_tokens // nw
    chunk = 32
    n_chunks = per_w // chunk
    mesh = plsc.VectorSubcoreMesh(core_axis_name="c", subcore_axis_name="s")

    @functools.partial(
        pl.kernel,
        out_type=jax.ShapeDtypeStruct((num_tokens, d_model), jnp.float32),
        mesh=mesh,
        scratch_types=[
            pltpu.VMEM((per_w,), jnp.int32),
            pltpu.VMEM((chunk, d_model), jnp.float32),
            pltpu.VMEM((chunk, d_model), jnp.float32),
            pltpu.SemaphoreType.DMA,
        ],
    )
    def emb(x_hbm, tok_hbm, pos_hbm, out_hbm, idx_v, rows_v, pos_v, sem):
        wid = lax.axis_index("s") * nc + lax.axis_index("c")
        base = wid * per_w
        pos_base = lax.rem(base, seq_len)
        pltpu.sync_copy(x_hbm.at[pl.ds(base, per_w)], idx_v)
        for ch in range(n_chunks):
            gather = pltpu.async_copy(
                tok_hbm.at[idx_v.at[pl.ds(ch * chunk, chunk)]], rows_v, sem
            )
            pltpu.sync_copy(pos_hbm.at[pl.ds(pos_base + ch * chunk, chunk)], pos_v)
            gather.wait()

            @plsc.parallel_loop(0, chunk * d_model, _LANES, unroll=8)
            def _(i):
                r = i // d_model
                col = i % d_model
                rows_v[r, pl.ds(col, _LANES)] = (
                    rows_v[r, pl.ds(col, _LANES)] + pos_v[r, pl.ds(col, _LANES)]
                )

            pltpu.sync_copy(rows_v, out_hbm.at[pl.ds(base + ch * chunk, chunk)])

    return emb


def kernel(x, token_table, pos_table):
    batch, seq_len = x.shape
    d_model = token_table.shape[1]
    emb = _build(batch * seq_len, seq_len, d_model)
    out = emb(x.reshape(-1).astype(jnp.int32), token_table, pos_table)
    return out.reshape(batch, seq_len, d_model)


# speedup vs baseline: 1.3089x; 1.2719x over previous
import functools

import jax
import jax.numpy as jnp
from jax import lax
from jax.experimental import pallas as pl
from jax.experimental.pallas import tpu as pltpu
from jax.experimental.pallas import tpu_sc as plsc


_LANES = 16


@functools.cache
def _build(num_tokens: int, seq_len: int, d_model: int):
    info = plsc.get_sparse_core_info()
    nc, ns = info.num_cores, info.num_subcores
    nw = nc * ns
    per_w = num_tokens // nw
    chunk = 16
    n_chunks = per_w // chunk
    mesh = plsc.VectorSubcoreMesh(core_axis_name="c", subcore_axis_name="s")

    @functools.partial(
        pl.kernel,
        out_type=jax.ShapeDtypeStruct((num_tokens, d_model), jnp.float32),
        mesh=mesh,
        scratch_types=[
            pltpu.VMEM((per_w,), jnp.int32),
            pltpu.VMEM((chunk, d_model), jnp.float32),
            pltpu.VMEM((chunk, d_model), jnp.float32),
            pltpu.VMEM((chunk, d_model), jnp.float32),
            pltpu.VMEM((chunk, d_model), jnp.float32),
            pltpu.SemaphoreType.DMA,
            pltpu.SemaphoreType.DMA,
            pltpu.SemaphoreType.DMA,
            pltpu.SemaphoreType.DMA,
            pltpu.SemaphoreType.DMA,
            pltpu.SemaphoreType.DMA,
        ],
    )
    def emb(x_hbm, tok_hbm, pos_hbm, out_hbm, idx_v,
            rows0, rows1, pos0, pos1, gs0, gs1, ps0, ps1, ss0, ss1):
        rows = (rows0, rows1)
        pos = (pos0, pos1)
        gsem = (gs0, gs1)
        psem = (ps0, ps1)
        ssem = (ss0, ss1)
        wid = lax.axis_index("s") * nc + lax.axis_index("c")
        base = wid * per_w
        pos_base = lax.rem(base, seq_len)
        pltpu.sync_copy(x_hbm.at[pl.ds(base, per_w)], idx_v)

        def start(ch):
            b = ch % 2
            g = pltpu.async_copy(
                tok_hbm.at[idx_v.at[pl.ds(ch * chunk, chunk)]], rows[b], gsem[b]
            )
            p = pltpu.async_copy(
                pos_hbm.at[pl.ds(pos_base + ch * chunk, chunk)], pos[b], psem[b]
            )
            return g, p

        inflight = {0: start(0)}
        stores = {}
        for ch in range(n_chunks):
            b = ch % 2
            if ch + 1 < n_chunks:
                if ch - 1 in stores:
                    stores.pop(ch - 1).wait()
                inflight[ch + 1] = start(ch + 1)
            g, p = inflight.pop(ch)
            g.wait()
            p.wait()

            @plsc.parallel_loop(0, chunk * d_model, _LANES, unroll=8)
            def _(i):
                r = i // d_model
                col = i % d_model
                rows[b][r, pl.ds(col, _LANES)] = (
                    rows[b][r, pl.ds(col, _LANES)] + pos[b][r, pl.ds(col, _LANES)]
                )

            stores[ch] = pltpu.async_copy(
                rows[b], out_hbm.at[pl.ds(base + ch * chunk, chunk)], ssem[b]
            )
        for ch in sorted(stores):
            stores.pop(ch).wait()

    return emb


def kernel(x, token_table, pos_table):
    batch, seq_len = x.shape
    d_model = token_table.shape[1]
    emb = _build(batch * seq_len, seq_len, d_model)
    out = emb(x.reshape(-1).astype(jnp.int32), token_table, pos_table)
    return out.reshape(batch, seq_len, d_model)
